# baseline (speedup 1.0000x reference)
import functools

import jax
import jax.numpy as jnp
from jax import lax
from jax.experimental import pallas as pl
from jax.experimental.pallas import tpu as pltpu
from jax.experimental.pallas import tpu_sc as plsc

H, W, FD = 1024, 1024, 2
W2 = W + 1
NC, NS, L = 2, 16, 16
NW = NC * NS

N = 16384 * 200
NP = 200
NX = 16384
SEG = NX // NW

PV = 16 * 66560
STAGE = PV // NS


def _idx_loop(xy_v, i00_v, i01_v, i10_v, i11_v, wx_v, wy_v):
    def idx_body(i, carry):
        b = i * L
        xo = (i // 8) * 256 + (i % 8) * L
        x = xy_v[pl.ds(xo, L)] * float(W - 1)
        y = xy_v[pl.ds(xo + 128, L)] * float(H - 1)
        x0 = x.astype(jnp.int32)
        y0 = y.astype(jnp.int32)
        wx = x - x0.astype(jnp.float32)
        wy = y - y0.astype(jnp.float32)
        idx = y0 * W2 + x0
        i00_v[pl.ds(b, L)] = idx
        i01_v[pl.ds(b, L)] = idx + 1
        i10_v[pl.ds(b, L)] = idx + W2
        i11_v[pl.ds(b, L)] = idx + (W2 + 1)
        wx_v[pl.ds(b, L)] = wx
        wy_v[pl.ds(b, L)] = wy
        return carry

    lax.fori_loop(0, SEG // L, idx_body, 0)


def _mix_loop(r00_v, r01_v, r10_v, r11_v, wx_v, wy_v, o_v):
    def mix_body(i, carry):
        b = i * L
        wx = wx_v[pl.ds(b, L)]
        wy = wy_v[pl.ds(b, L)]
        u00 = plsc.bitcast(r00_v[pl.ds(b, L)], jnp.int32)
        u01 = plsc.bitcast(r01_v[pl.ds(b, L)], jnp.int32)
        u10 = plsc.bitcast(r10_v[pl.ds(b, L)], jnp.int32)
        u11 = plsc.bitcast(r11_v[pl.ds(b, L)], jnp.int32)
        hm = jnp.int32(-65536)
        a00 = plsc.bitcast(lax.shift_left(u00, 16), jnp.float32)
        a01 = plsc.bitcast(lax.shift_left(u01, 16), jnp.float32)
        a10 = plsc.bitcast(lax.shift_left(u10, 16), jnp.float32)
        a11 = plsc.bitcast(lax.shift_left(u11, 16), jnp.float32)
        b00 = plsc.bitcast(u00 & hm, jnp.float32)
        b01 = plsc.bitcast(u01 & hm, jnp.float32)
        b10 = plsc.bitcast(u10 & hm, jnp.float32)
        b11 = plsc.bitcast(u11 & hm, jnp.float32)
        t0 = a00 + wx * (a01 - a00)
        u0 = a10 + wx * (a11 - a10)
        t1 = b00 + wx * (b01 - b00)
        u1 = b10 + wx * (b11 - b10)
        oo = (i // 8) * 256 + (i % 8) * L
        o_v[pl.ds(oo, L)] = t0 + wy * (u0 - t0)
        o_v[pl.ds(oo + 128, L)] = t1 + wy * (u1 - t1)
        return carry

    lax.fori_loop(0, SEG // L, mix_body, 0)


def _sc_body(coords_hbm, table_hbm, out_hbm, shared,
             xy0_v, xy1_v,
             a00_v, a01_v, a10_v, a11_v, b00_v, b01_v, b10_v, b11_v,
             wxa_v, wya_v, wxb_v, wyb_v,
             p00_v, p01_v, p10_v, p11_v, q00_v, q01_v, q10_v, q11_v,
             oa_v, ob_v,
             si0, si1, sg0, sg1, so0, so1):
    cid = lax.axis_index("c")
    sid = lax.axis_index("s")
    wid = sid * NC + cid
    nbase = wid * (2 * SEG)

    xys = [xy0_v, xy1_v]
    idxs = [[a00_v, a01_v, a10_v, a11_v], [b00_v, b01_v, b10_v, b11_v]]
    wxs = [wxa_v, wxb_v]
    wys = [wya_v, wyb_v]
    rs = [[p00_v, p01_v, p10_v, p11_v], [q00_v, q01_v, q10_v, q11_v]]
    os_ = [oa_v, ob_v]
    sin = [si0, si1]
    sgat = [sg0, sg1]
    sout = [so0, so1]

    pltpu.sync_copy(table_hbm.at[pl.ds(sid * STAGE, STAGE)],
                    shared.at[pl.ds(sid * STAGE, STAGE)])
    plsc.subcore_barrier()

    def in_start(k, b):
        base = k * (2 * NX) + nbase
        pltpu.async_copy(coords_hbm.at[pl.ds(base, 2 * SEG)], xys[b], sin[b])

    def in_wait(k, b):
        base = k * (2 * NX) + nbase
        pltpu.make_async_copy(coords_hbm.at[pl.ds(base, 2 * SEG)], xys[b], sin[b]).wait()

    def gat_start(b):
        for iv, rv in zip(idxs[b], rs[b]):
            pltpu.async_copy(shared.at[iv], rv, sgat[b])

    def gat_wait(b):
        for iv, rv in zip(idxs[b], rs[b]):
            pltpu.make_async_copy(shared.at[iv], rv, sgat[b]).wait()

    def out_start(k, b):
        base = k * (2 * NX) + nbase
        pltpu.async_copy(os_[b], out_hbm.at[pl.ds(base, 2 * SEG)], sout[b])

    def out_wait(k, b):
        base = k * (2 * NX) + nbase
        pltpu.make_async_copy(os_[b], out_hbm.at[pl.ds(base, 2 * SEG)], sout[b]).wait()

    in_start(0, 0)

    def pipe_body(t, carry):
        for buf in (0, 1):
            k = t * 2 + buf
            nbuf = 1 - buf
            if buf == 0:
                in_start(k + 1, nbuf)
            else:
                @pl.when(k + 1 < NP)
                def _():
                    in_start(k + 1, nbuf)
            in_wait(k, buf)
            _idx_loop(xys[buf], *idxs[buf], wxs[buf], wys[buf])
            gat_start(buf)

            def tail():
                gat_wait(nbuf)

                @pl.when(k >= 3)
                def _():
                    out_wait(k - 3, nbuf)

                _mix_loop(*rs[nbuf], wxs[nbuf], wys[nbuf], os_[nbuf])
                out_start(k - 1, nbuf)

            if buf == 1:
                tail()
            else:
                @pl.when(k >= 1)
                def _():
                    tail()
        return carry

    lax.fori_loop(0, NP // 2, pipe_body, 0)

    gat_wait(1)
    out_wait(NP - 3, 1)
    _mix_loop(*rs[1], wxs[1], wys[1], os_[1])
    out_start(NP - 1, 1)
    out_wait(NP - 2, 0)
    out_wait(NP - 1, 1)


_sc_sample = functools.partial(
    pl.kernel,
    out_type=jax.ShapeDtypeStruct((N * FD,), jnp.float32),
    mesh=plsc.VectorSubcoreMesh(
        core_axis_name="c", subcore_axis_name="s", num_cores=NC, num_subcores=NS
    ),
    compiler_params=pltpu.CompilerParams(needs_layout_passes=False),
    scratch_types=[
        pltpu.VMEM_SHARED((PV,), jnp.float32),
    ] + [pltpu.VMEM((2 * SEG,), jnp.float32) for _ in range(2)]
    + [pltpu.VMEM((SEG,), jnp.int32) for _ in range(8)]
    + [pltpu.VMEM((SEG,), jnp.float32) for _ in range(4)]
    + [pltpu.VMEM((SEG,), jnp.float32) for _ in range(8)]
    + [pltpu.VMEM((2 * SEG,), jnp.float32) for _ in range(2)]
    + [pltpu.SemaphoreType.DMA for _ in range(6)],
)(_sc_body)


def kernel(coords, vector_field):
    vt = vector_field.transpose(0, 2, 1)
    g16 = lax.bitcast_convert_type(
        vt.astype(jnp.bfloat16), jnp.uint16
    ).astype(jnp.uint32)
    packed = g16[:, 0, :] | (g16[:, 1, :] << 16)
    packed = jnp.pad(packed, ((0, 1), (0, 1)), mode="edge").reshape(-1)
    packed = jnp.pad(packed, (0, PV - W2 * (H + 1)))
    table = lax.bitcast_convert_type(packed, jnp.float32)
    ct = coords.reshape(NX // 128, 128, NP, FD).transpose(2, 0, 3, 1).reshape(-1)
    out = _sc_sample(ct, table)
    return (out.reshape(NP, NX // 128, FD, 128)
            .transpose(1, 3, 0, 2).reshape(NX, NP, FD))


# speedup vs baseline: 1.0783x; 1.0783x over previous
import functools

import jax
import jax.numpy as jnp
from jax import lax
from jax.experimental import pallas as pl
from jax.experimental.pallas import tpu as pltpu
from jax.experimental.pallas import tpu_sc as plsc

H, W, FD = 1024, 1024, 2
NC, NS, L = 2, 16, 16
NW = NC * NS

N = 16384 * 200
NP = 200
NX = 16384
SEG = NX // NW

PV = H * W
STAGE = PV // NS


def _idx_loop(xy_v, i00_v, i01_v, i10_v, i11_v, wx_v, wy_v):
    def idx_body(i, carry):
        b = i * L
        xo = (i // 8) * 256 + (i % 8) * L
        x = xy_v[pl.ds(xo, L)] * float(W - 1)
        y = xy_v[pl.ds(xo + 128, L)] * float(H - 1)
        x0 = x.astype(jnp.int32)
        y0 = y.astype(jnp.int32)
        wx = x - x0.astype(jnp.float32)
        wy = y - y0.astype(jnp.float32)
        idx = lax.shift_left(y0, 10) + x0
        i00_v[pl.ds(b, L)] = idx
        i01_v[pl.ds(b, L)] = idx + 1
        i10_v[pl.ds(b, L)] = idx + W
        i11_v[pl.ds(b, L)] = idx + (W + 1)
        wx_v[pl.ds(b, L)] = wx
        wy_v[pl.ds(b, L)] = wy
        return carry

    lax.fori_loop(0, SEG // L, idx_body, 0)


def _mix_loop(r00_v, r01_v, r10_v, r11_v, wx_v, wy_v, o_v):
    def mix_body(i, carry):
        b = i * L
        wx = wx_v[pl.ds(b, L)]
        wy = wy_v[pl.ds(b, L)]
        u00 = plsc.bitcast(r00_v[pl.ds(b, L)], jnp.int32)
        u01 = plsc.bitcast(r01_v[pl.ds(b, L)], jnp.int32)
        u10 = plsc.bitcast(r10_v[pl.ds(b, L)], jnp.int32)
        u11 = plsc.bitcast(r11_v[pl.ds(b, L)], jnp.int32)
        hm = jnp.int32(-65536)
        a00 = plsc.bitcast(lax.shift_left(u00, 16), jnp.float32)
        a01 = plsc.bitcast(lax.shift_left(u01, 16), jnp.float32)
        a10 = plsc.bitcast(lax.shift_left(u10, 16), jnp.float32)
        a11 = plsc.bitcast(lax.shift_left(u11, 16), jnp.float32)
        b00 = plsc.bitcast(u00 & hm, jnp.float32)
        b01 = plsc.bitcast(u01 & hm, jnp.float32)
        b10 = plsc.bitcast(u10 & hm, jnp.float32)
        b11 = plsc.bitcast(u11 & hm, jnp.float32)
        t0 = a00 + wx * (a01 - a00)
        u0 = a10 + wx * (a11 - a10)
        t1 = b00 + wx * (b01 - b00)
        u1 = b10 + wx * (b11 - b10)
        oo = (i // 8) * 256 + (i % 8) * L
        o_v[pl.ds(oo, L)] = t0 + wy * (u0 - t0)
        o_v[pl.ds(oo + 128, L)] = t1 + wy * (u1 - t1)
        return carry

    lax.fori_loop(0, SEG // L, mix_body, 0)


def _sc_body(coords_hbm, table_hbm, out_hbm, shared,
             xy0_v, xy1_v,
             a00_v, a01_v, a10_v, a11_v, b00_v, b01_v, b10_v, b11_v,
             wxa_v, wya_v, wxb_v, wyb_v,
             p00_v, p01_v, p10_v, p11_v, q00_v, q01_v, q10_v, q11_v,
             oa_v, ob_v,
             si0, si1, sg0, sg1, so0, so1):
    cid = lax.axis_index("c")
    sid = lax.axis_index("s")
    wid = sid * NC + cid
    nbase = wid * (2 * SEG)

    xys = [xy0_v, xy1_v]
    idxs = [[a00_v, a01_v, a10_v, a11_v], [b00_v, b01_v, b10_v, b11_v]]
    wxs = [wxa_v, wxb_v]
    wys = [wya_v, wyb_v]
    rs = [[p00_v, p01_v, p10_v, p11_v], [q00_v, q01_v, q10_v, q11_v]]
    os_ = [oa_v, ob_v]
    sin = [si0, si1]
    sgat = [sg0, sg1]
    sout = [so0, so1]

    pltpu.sync_copy(table_hbm.at[pl.ds(sid * STAGE, STAGE)],
                    shared.at[pl.ds(sid * STAGE, STAGE)])
    plsc.subcore_barrier()

    def in_start(k, b):
        base = k * (2 * NX) + nbase
        pltpu.async_copy(coords_hbm.at[pl.ds(base, 2 * SEG)], xys[b], sin[b])

    def in_wait(k, b):
        base = k * (2 * NX) + nbase
        pltpu.make_async_copy(coords_hbm.at[pl.ds(base, 2 * SEG)], xys[b], sin[b]).wait()

    def gat_start(b):
        for iv, rv in zip(idxs[b], rs[b]):
            pltpu.async_copy(shared.at[iv], rv, sgat[b])

    def gat_wait(b):
        for iv, rv in zip(idxs[b], rs[b]):
            pltpu.make_async_copy(shared.at[iv], rv, sgat[b]).wait()

    def out_start(k, b):
        base = k * (2 * NX) + nbase
        pltpu.async_copy(os_[b], out_hbm.at[pl.ds(base, 2 * SEG)], sout[b])

    def out_wait(k, b):
        base = k * (2 * NX) + nbase
        pltpu.make_async_copy(os_[b], out_hbm.at[pl.ds(base, 2 * SEG)], sout[b]).wait()

    in_start(0, 0)

    def pipe_body(t, carry):
        for buf in (0, 1):
            k = t * 2 + buf
            nbuf = 1 - buf
            if buf == 0:
                in_start(k + 1, nbuf)
            else:
                @pl.when(k + 1 < NP)
                def _():
                    in_start(k + 1, nbuf)
            in_wait(k, buf)
            _idx_loop(xys[buf], *idxs[buf], wxs[buf], wys[buf])
            gat_start(buf)

            def tail():
                gat_wait(nbuf)

                @pl.when(k >= 3)
                def _():
                    out_wait(k - 3, nbuf)

                _mix_loop(*rs[nbuf], wxs[nbuf], wys[nbuf], os_[nbuf])
                out_start(k - 1, nbuf)

            if buf == 1:
                tail()
            else:
                @pl.when(k >= 1)
                def _():
                    tail()
        return carry

    lax.fori_loop(0, NP // 2, pipe_body, 0)

    gat_wait(1)
    out_wait(NP - 3, 1)
    _mix_loop(*rs[1], wxs[1], wys[1], os_[1])
    out_start(NP - 1, 1)
    out_wait(NP - 2, 0)
    out_wait(NP - 1, 1)


_sc_sample = functools.partial(
    pl.kernel,
    out_type=jax.ShapeDtypeStruct((N * FD,), jnp.float32),
    mesh=plsc.VectorSubcoreMesh(
        core_axis_name="c", subcore_axis_name="s", num_cores=NC, num_subcores=NS
    ),
    compiler_params=pltpu.CompilerParams(needs_layout_passes=False),
    scratch_types=[
        pltpu.VMEM_SHARED((PV,), jnp.float32),
    ] + [pltpu.VMEM((2 * SEG,), jnp.float32) for _ in range(2)]
    + [pltpu.VMEM((SEG,), jnp.int32) for _ in range(8)]
    + [pltpu.VMEM((SEG,), jnp.float32) for _ in range(4)]
    + [pltpu.VMEM((SEG,), jnp.float32) for _ in range(8)]
    + [pltpu.VMEM((2 * SEG,), jnp.float32) for _ in range(2)]
    + [pltpu.SemaphoreType.DMA for _ in range(6)],
)(_sc_body)


def kernel(coords, vector_field):
    vt = vector_field.transpose(0, 2, 1)
    g16 = lax.bitcast_convert_type(
        vt.astype(jnp.bfloat16), jnp.uint16
    ).astype(jnp.uint32)
    packed = (g16[:, 0, :] | (g16[:, 1, :] << 16)).reshape(-1)
    table = lax.bitcast_convert_type(packed, jnp.float32)
    ct = coords.reshape(NX // 128, 128, NP, FD).transpose(2, 0, 3, 1).reshape(-1)
    out = _sc_sample(ct, table)
    return (out.reshape(NP, NX // 128, FD, 128)
            .transpose(1, 3, 0, 2).reshape(NX, NP, FD))


# speedup vs baseline: 1.0857x; 1.0068x over previous
import functools

import jax
import jax.numpy as jnp
from jax import lax
from jax.experimental import pallas as pl
from jax.experimental.pallas import tpu as pltpu
from jax.experimental.pallas import tpu_sc as plsc

H, W, FD = 1024, 1024, 2
NC, NS, L = 2, 16, 16
NW = NC * NS

N = 16384 * 200
UNROLL = 2
NP = 200
NX = 16384
SEG = NX // NW

PV = H * W
STAGE = PV // NS


def _idx_loop(xy_v, i00_v, i01_v, i10_v, i11_v, wx_v, wy_v):
    def idx_body(t, carry):
        for u in range(UNROLL):
            i = t * UNROLL + u
            b = i * L
            xo = (i // 8) * 256 + (i % 8) * L
            x = xy_v[pl.ds(xo, L)] * float(W - 1)
            y = xy_v[pl.ds(xo + 128, L)] * float(H - 1)
            x0 = x.astype(jnp.int32)
            y0 = y.astype(jnp.int32)
            wx = x - x0.astype(jnp.float32)
            wy = y - y0.astype(jnp.float32)
            idx = lax.shift_left(y0, 10) + x0
            i00_v[pl.ds(b, L)] = idx
            i01_v[pl.ds(b, L)] = idx + 1
            i10_v[pl.ds(b, L)] = idx + W
            i11_v[pl.ds(b, L)] = idx + (W + 1)
            wx_v[pl.ds(b, L)] = wx
            wy_v[pl.ds(b, L)] = wy
        return carry

    lax.fori_loop(0, SEG // L // UNROLL, idx_body, 0)


def _mix_loop(r00_v, r01_v, r10_v, r11_v, wx_v, wy_v, o_v):
    def mix_body(t, carry):
      for u in range(UNROLL):
        i = t * UNROLL + u
        b = i * L
        wx = wx_v[pl.ds(b, L)]
        wy = wy_v[pl.ds(b, L)]
        u00 = plsc.bitcast(r00_v[pl.ds(b, L)], jnp.int32)
        u01 = plsc.bitcast(r01_v[pl.ds(b, L)], jnp.int32)
        u10 = plsc.bitcast(r10_v[pl.ds(b, L)], jnp.int32)
        u11 = plsc.bitcast(r11_v[pl.ds(b, L)], jnp.int32)
        hm = jnp.int32(-65536)
        a00 = plsc.bitcast(lax.shift_left(u00, 16), jnp.float32)
        a01 = plsc.bitcast(lax.shift_left(u01, 16), jnp.float32)
        a10 = plsc.bitcast(lax.shift_left(u10, 16), jnp.float32)
        a11 = plsc.bitcast(lax.shift_left(u11, 16), jnp.float32)
        b00 = plsc.bitcast(u00 & hm, jnp.float32)
        b01 = plsc.bitcast(u01 & hm, jnp.float32)
        b10 = plsc.bitcast(u10 & hm, jnp.float32)
        b11 = plsc.bitcast(u11 & hm, jnp.float32)
        t0 = a00 + wx * (a01 - a00)
        u0 = a10 + wx * (a11 - a10)
        t1 = b00 + wx * (b01 - b00)
        u1 = b10 + wx * (b11 - b10)
        oo = (i // 8) * 256 + (i % 8) * L
        o_v[pl.ds(oo, L)] = t0 + wy * (u0 - t0)
        o_v[pl.ds(oo + 128, L)] = t1 + wy * (u1 - t1)
      return carry

    lax.fori_loop(0, SEG // L // UNROLL, mix_body, 0)


def _sc_body(coords_hbm, table_hbm, out_hbm, shared,
             xy0_v, xy1_v,
             a00_v, a01_v, a10_v, a11_v, b00_v, b01_v, b10_v, b11_v,
             wxa_v, wya_v, wxb_v, wyb_v,
             p00_v, p01_v, p10_v, p11_v, q00_v, q01_v, q10_v, q11_v,
             oa_v, ob_v,
             si0, si1, sg0, sg1, so0, so1):
    cid = lax.axis_index("c")
    sid = lax.axis_index("s")
    wid = sid * NC + cid
    nbase = wid * (2 * SEG)

    xys = [xy0_v, xy1_v]
    idxs = [[a00_v, a01_v, a10_v, a11_v], [b00_v, b01_v, b10_v, b11_v]]
    wxs = [wxa_v, wxb_v]
    wys = [wya_v, wyb_v]
    rs = [[p00_v, p01_v, p10_v, p11_v], [q00_v, q01_v, q10_v, q11_v]]
    os_ = [oa_v, ob_v]
    sin = [si0, si1]
    sgat = [sg0, sg1]
    sout = [so0, so1]

    pltpu.sync_copy(table_hbm.at[pl.ds(sid * STAGE, STAGE)],
                    shared.at[pl.ds(sid * STAGE, STAGE)])
    plsc.subcore_barrier()

    def in_start(k, b):
        base = k * (2 * NX) + nbase
        pltpu.async_copy(coords_hbm.at[pl.ds(base, 2 * SEG)], xys[b], sin[b])

    def in_wait(k, b):
        base = k * (2 * NX) + nbase
        pltpu.make_async_copy(coords_hbm.at[pl.ds(base, 2 * SEG)], xys[b], sin[b]).wait()

    def gat_start(b):
        for iv, rv in zip(idxs[b], rs[b]):
            pltpu.async_copy(shared.at[iv], rv, sgat[b])

    def gat_wait(b):
        for iv, rv in zip(idxs[b], rs[b]):
            pltpu.make_async_copy(shared.at[iv], rv, sgat[b]).wait()

    def out_start(k, b):
        base = k * (2 * NX) + nbase
        pltpu.async_copy(os_[b], out_hbm.at[pl.ds(base, 2 * SEG)], sout[b])

    def out_wait(k, b):
        base = k * (2 * NX) + nbase
        pltpu.make_async_copy(os_[b], out_hbm.at[pl.ds(base, 2 * SEG)], sout[b]).wait()

    in_start(0, 0)

    def pipe_body(t, carry):
        for buf in (0, 1):
            k = t * 2 + buf
            nbuf = 1 - buf
            if buf == 0:
                in_start(k + 1, nbuf)
            else:
                @pl.when(k + 1 < NP)
                def _():
                    in_start(k + 1, nbuf)
            in_wait(k, buf)
            _idx_loop(xys[buf], *idxs[buf], wxs[buf], wys[buf])
            gat_start(buf)

            def tail():
                gat_wait(nbuf)

                @pl.when(k >= 3)
                def _():
                    out_wait(k - 3, nbuf)

                _mix_loop(*rs[nbuf], wxs[nbuf], wys[nbuf], os_[nbuf])
                out_start(k - 1, nbuf)

            if buf == 1:
                tail()
            else:
                @pl.when(k >= 1)
                def _():
                    tail()
        return carry

    lax.fori_loop(0, NP // 2, pipe_body, 0)

    gat_wait(1)
    out_wait(NP - 3, 1)
    _mix_loop(*rs[1], wxs[1], wys[1], os_[1])
    out_start(NP - 1, 1)
    out_wait(NP - 2, 0)
    out_wait(NP - 1, 1)


_sc_sample = functools.partial(
    pl.kernel,
    out_type=jax.ShapeDtypeStruct((N * FD,), jnp.float32),
    mesh=plsc.VectorSubcoreMesh(
        core_axis_name="c", subcore_axis_name="s", num_cores=NC, num_subcores=NS
    ),
    compiler_params=pltpu.CompilerParams(needs_layout_passes=False),
    scratch_types=[
        pltpu.VMEM_SHARED((PV,), jnp.float32),
    ] + [pltpu.VMEM((2 * SEG,), jnp.float32) for _ in range(2)]
    + [pltpu.VMEM((SEG,), jnp.int32) for _ in range(8)]
    + [pltpu.VMEM((SEG,), jnp.float32) for _ in range(4)]
    + [pltpu.VMEM((SEG,), jnp.float32) for _ in range(8)]
    + [pltpu.VMEM((2 * SEG,), jnp.float32) for _ in range(2)]
    + [pltpu.SemaphoreType.DMA for _ in range(6)],
)(_sc_body)


def kernel(coords, vector_field):
    vt = vector_field.transpose(0, 2, 1)
    g16 = lax.bitcast_convert_type(
        vt.astype(jnp.bfloat16), jnp.uint16
    ).astype(jnp.uint32)
    packed = (g16[:, 0, :] | (g16[:, 1, :] << 16)).reshape(-1)
    table = lax.bitcast_convert_type(packed, jnp.float32)
    ct = coords.reshape(NX // 128, 128, NP, FD).transpose(2, 0, 3, 1).reshape(-1)
    out = _sc_sample(ct, table)
    return (out.reshape(NP, NX // 128, FD, 128)
            .transpose(1, 3, 0, 2).reshape(NX, NP, FD))


# speedup vs baseline: 1.0986x; 1.0119x over previous
import functools

import jax
import jax.numpy as jnp
from jax import lax
from jax.experimental import pallas as pl
from jax.experimental.pallas import tpu as pltpu
from jax.experimental.pallas import tpu_sc as plsc

H, W, FD = 1024, 1024, 2
NC, NS, L = 2, 16, 16
NW = NC * NS

N = 16384 * 200
UNROLL = 2
NP = 200
NX = 16384
SEG = NX // NW

PV = H * W
STAGE = PV // NS


def _idx_loop(xy_v, i00_v, i01_v, wx_v, wy_v):
    def idx_body(t, carry):
        for u in range(UNROLL):
            i = t * UNROLL + u
            b = i * L
            xo = (i // 8) * 256 + (i % 8) * L
            x = xy_v[pl.ds(xo, L)] * float(W - 1)
            y = xy_v[pl.ds(xo + 128, L)] * float(H - 1)
            x0 = x.astype(jnp.int32)
            y0 = y.astype(jnp.int32)
            wx = x - x0.astype(jnp.float32)
            wy = y - y0.astype(jnp.float32)
            idx = lax.shift_left(y0, 10) + x0
            i00_v[pl.ds(b, L)] = idx
            i01_v[pl.ds(b, L)] = idx + 1
            wx_v[pl.ds(b, L)] = wx
            wy_v[pl.ds(b, L)] = wy
        return carry

    lax.fori_loop(0, SEG // L // UNROLL, idx_body, 0)


def _mix_loop(r00_v, r01_v, r10_v, r11_v, wx_v, wy_v, o_v):
    def mix_body(t, carry):
      for u in range(UNROLL):
        i = t * UNROLL + u
        b = i * L
        wx = wx_v[pl.ds(b, L)]
        wy = wy_v[pl.ds(b, L)]
        wx2 = plsc.pack(wx, wx, format=plsc.PackFormat.INTERLEAVED)
        wy2 = plsc.pack(wy, wy, format=plsc.PackFormat.INTERLEAVED)
        g00 = plsc.bitcast(r00_v[pl.ds(b, L)], jnp.bfloat16)
        g01 = plsc.bitcast(r01_v[pl.ds(b, L)], jnp.bfloat16)
        g10 = plsc.bitcast(r10_v[pl.ds(b, L)], jnp.bfloat16)
        g11 = plsc.bitcast(r11_v[pl.ds(b, L)], jnp.bfloat16)
        top = g00 + wx2 * (g01 - g00)
        bot = g10 + wx2 * (g11 - g10)
        res = top + wy2 * (bot - top)
        o0, o1 = plsc.unpack(res, format=plsc.PackFormat.INTERLEAVED)
        oo = (i // 8) * 256 + (i % 8) * L
        o_v[pl.ds(oo, L)] = o0
        o_v[pl.ds(oo + 128, L)] = o1
      return carry

    lax.fori_loop(0, SEG // L // UNROLL, mix_body, 0)


def _sc_body(coords_hbm, table_hbm, out_hbm, shared,
             xy0_v, xy1_v,
             a00_v, a01_v, b00_v, b01_v,
             wxa_v, wya_v, wxb_v, wyb_v,
             p00_v, p01_v, p10_v, p11_v, q00_v, q01_v, q10_v, q11_v,
             oa_v, ob_v,
             si0, si1, sg0, sg1, so0, so1):
    cid = lax.axis_index("c")
    sid = lax.axis_index("s")
    wid = sid * NC + cid
    nbase = wid * (2 * SEG)

    xys = [xy0_v, xy1_v]
    idxs = [[a00_v, a01_v], [b00_v, b01_v]]
    wxs = [wxa_v, wxb_v]
    wys = [wya_v, wyb_v]
    rs = [[p00_v, p01_v, p10_v, p11_v], [q00_v, q01_v, q10_v, q11_v]]
    os_ = [oa_v, ob_v]
    sin = [si0, si1]
    sgat = [sg0, sg1]
    sout = [so0, so1]

    pltpu.sync_copy(table_hbm.at[pl.ds(sid * STAGE, STAGE)],
                    shared.at[pl.ds(sid * STAGE, STAGE)])
    plsc.subcore_barrier()

    def in_start(k, b):
        base = k * (2 * NX) + nbase
        pltpu.async_copy(coords_hbm.at[pl.ds(base, 2 * SEG)], xys[b], sin[b])

    def in_wait(k, b):
        base = k * (2 * NX) + nbase
        pltpu.make_async_copy(coords_hbm.at[pl.ds(base, 2 * SEG)], xys[b], sin[b]).wait()

    shifted = shared.at[pl.ds(W, PV - W)]

    def gat_start(b):
        i0, i1 = idxs[b]
        r00, r01, r10, r11 = rs[b]
        pltpu.async_copy(shared.at[i0], r00, sgat[b])
        pltpu.async_copy(shared.at[i1], r01, sgat[b])
        pltpu.async_copy(shifted.at[i0], r10, sgat[b])
        pltpu.async_copy(shifted.at[i1], r11, sgat[b])

    def gat_wait(b):
        i0, i1 = idxs[b]
        r00, r01, r10, r11 = rs[b]
        pltpu.make_async_copy(shared.at[i0], r00, sgat[b]).wait()
        pltpu.make_async_copy(shared.at[i1], r01, sgat[b]).wait()
        pltpu.make_async_copy(shifted.at[i0], r10, sgat[b]).wait()
        pltpu.make_async_copy(shifted.at[i1], r11, sgat[b]).wait()

    def out_start(k, b):
        base = k * (2 * NX) + nbase
        pltpu.async_copy(os_[b], out_hbm.at[pl.ds(base, 2 * SEG)], sout[b])

    def out_wait(k, b):
        base = k * (2 * NX) + nbase
        pltpu.make_async_copy(os_[b], out_hbm.at[pl.ds(base, 2 * SEG)], sout[b]).wait()

    in_start(0, 0)

    def pipe_body(t, carry):
        for buf in (0, 1):
            k = t * 2 + buf
            nbuf = 1 - buf
            if buf == 0:
                in_start(k + 1, nbuf)
            else:
                @pl.when(k + 1 < NP)
                def _():
                    in_start(k + 1, nbuf)
            in_wait(k, buf)
            _idx_loop(xys[buf], *idxs[buf], wxs[buf], wys[buf])
            gat_start(buf)

            def tail():
                gat_wait(nbuf)

                @pl.when(k >= 3)
                def _():
                    out_wait(k - 3, nbuf)

                _mix_loop(*rs[nbuf], wxs[nbuf], wys[nbuf], os_[nbuf])
                out_start(k - 1, nbuf)

            if buf == 1:
                tail()
            else:
                @pl.when(k >= 1)
                def _():
                    tail()
        return carry

    lax.fori_loop(0, NP // 2, pipe_body, 0)

    gat_wait(1)
    out_wait(NP - 3, 1)
    _mix_loop(*rs[1], wxs[1], wys[1], os_[1])
    out_start(NP - 1, 1)
    out_wait(NP - 2, 0)
    out_wait(NP - 1, 1)


_sc_sample = functools.partial(
    pl.kernel,
    out_type=jax.ShapeDtypeStruct((N * FD,), jnp.float32),
    mesh=plsc.VectorSubcoreMesh(
        core_axis_name="c", subcore_axis_name="s", num_cores=NC, num_subcores=NS
    ),
    compiler_params=pltpu.CompilerParams(needs_layout_passes=False),
    scratch_types=[
        pltpu.VMEM_SHARED((PV,), jnp.float32),
    ] + [pltpu.VMEM((2 * SEG,), jnp.float32) for _ in range(2)]
    + [pltpu.VMEM((SEG,), jnp.int32) for _ in range(4)]
    + [pltpu.VMEM((SEG,), jnp.float32) for _ in range(4)]
    + [pltpu.VMEM((SEG,), jnp.float32) for _ in range(8)]
    + [pltpu.VMEM((2 * SEG,), jnp.float32) for _ in range(2)]
    + [pltpu.SemaphoreType.DMA for _ in range(6)],
)(_sc_body)


def kernel(coords, vector_field):
    vt = vector_field.transpose(0, 2, 1)
    g16 = lax.bitcast_convert_type(
        vt.astype(jnp.bfloat16), jnp.uint16
    ).astype(jnp.uint32)
    packed = (g16[:, 0, :] | (g16[:, 1, :] << 16)).reshape(-1)
    table = lax.bitcast_convert_type(packed, jnp.float32)
    ct = coords.reshape(NX // 128, 128, NP, FD).transpose(2, 0, 3, 1).reshape(-1)
    out = _sc_sample(ct, table)
    return (out.reshape(NP, NX // 128, FD, 128)
            .transpose(1, 3, 0, 2).reshape(NX, NP, FD))
